# baseline (speedup 1.0000x reference)
import functools

import jax
import jax.numpy as jnp
from jax import lax
from jax.experimental import pallas as pl
from jax.experimental.pallas import tpu as pltpu
from jax.experimental.pallas import tpu_sc as plsc

_EPS = 1e-7
_LN2 = 0.6931471805599453
_SQRT2 = 1.4142135623730951

_N = 36864
_NS, _L = 16, 16
_NA = _N // _NS
_ND = 4 * _NA
_TILE = 2 * _ND + 2 * _NA
_OD0, _TD0, _OS0, _TS0 = 0, _ND, 2 * _ND, 2 * _ND + _NA
_PAC = _TILE


def _log16(x):
    bits = lax.bitcast_convert_type(x, jnp.int32)
    e = lax.shift_right_logical(bits, 23) - 127
    m = lax.bitcast_convert_type((bits & 0x007FFFFF) | 0x3F800000, jnp.float32)
    big = m > _SQRT2
    m = jnp.where(big, m * 0.5, m)
    e = e + jnp.where(big, 1, 0)
    s = (m - 1.0) / (m + 1.0)
    z = s * s
    p = 1.0 + z * (1.0 / 3.0 + z * (1.0 / 5.0 + z * (1.0 / 7.0 + z * (1.0 / 9.0))))
    return e.astype(jnp.float32) * _LN2 + 2.0 * s * p


@functools.partial(
    pl.kernel,
    out_type=jax.ShapeDtypeStruct((_NS * 4 * _L,), jnp.float32),
    mesh=plsc.VectorSubcoreMesh(
        core_axis_name="c", subcore_axis_name="s",
        num_cores=1, num_subcores=_NS,
    ),
    scratch_types=[
        pltpu.VMEM((_TILE + 4 * _L,), jnp.float32),
        pltpu.SemaphoreType.DMA,
    ],
)
def _sc_partials(inp_hbm, part_hbm, buf_v, sem):
    sid = lax.axis_index("s")
    pltpu.async_copy(
        inp_hbm.at[pl.ds(sid * _TILE, _TILE)],
        buf_v.at[pl.ds(0, _TILE)], sem).wait()

    zeros = jnp.zeros((_L,), jnp.float32)
    lane = lax.broadcasted_iota(jnp.int32, (_L,), 0)

    def body(k, carry):
        bce_a, nv_a, ps_a, reg_a = carry
        o_raw = buf_v[pl.ds(_OS0 + k * _L, _L)]
        t = buf_v[pl.ds(_TS0 + k * _L, _L)]
        o = jnp.clip(o_raw, _EPS, 1.0 - _EPS)
        bce = -(t * _log16(o) + (1.0 - t) * _log16(1.0 - o))
        valid = t != -1.0
        bce_a = bce_a + jnp.where(valid, bce, 0.0)
        nv_a = nv_a + jnp.where(valid, 1.0, 0.0)
        ps_a = ps_a + jnp.where(o_raw > 0.0, 1.0, 0.0)
        for v in range(4):
            dbase = k * (4 * _L) + v * _L
            od16 = buf_v[pl.ds(_OD0 + dbase, _L)]
            td16 = buf_v[pl.ds(_TD0 + dbase, _L)]
            d = jnp.abs(od16 - td16)
            sl1 = jnp.where(d < 1.0, 0.5 * d * d, d - 0.5)
            w = jnp.where(lane >= 12, o_raw[4 * v + 3],
                          jnp.where(lane >= 8, o_raw[4 * v + 2],
                                    jnp.where(lane >= 4, o_raw[4 * v + 1],
                                              o_raw[4 * v])))
            reg_a = reg_a + jnp.where(w > 0.0, sl1, 0.0)
        return bce_a, nv_a, ps_a, reg_a

    bce_a, nv_a, ps_a, reg_a = lax.fori_loop(
        0, _NA // _L, body, (zeros, zeros, zeros, zeros))

    buf_v[pl.ds(_PAC, _L)] = bce_a
    buf_v[pl.ds(_PAC + _L, _L)] = nv_a
    buf_v[pl.ds(_PAC + 2 * _L, _L)] = reg_a
    buf_v[pl.ds(_PAC + 3 * _L, _L)] = ps_a
    pltpu.sync_copy(
        buf_v.at[pl.ds(_PAC, 4 * _L)],
        part_hbm.at[pl.ds(sid * 4 * _L, 4 * _L)])


def _combine_body(p_ref, o_ref):
    p = p_ref[...]
    aid = lax.rem(lax.broadcasted_iota(jnp.int32, p.shape, 0), 4)
    bce = jnp.sum(jnp.where(aid == 0, p, 0.0))
    nv = jnp.sum(jnp.where(aid == 1, p, 0.0))
    reg = jnp.sum(jnp.where(aid == 2, p, 0.0))
    ps = jnp.sum(jnp.where(aid == 3, p, 0.0))
    cls_loss = bce / jnp.maximum(nv, 1.0)
    reg_loss = 10.0 * (reg / jnp.maximum(_EPS, ps))
    o_ref[0, 0] = cls_loss + reg_loss


_combine = pl.pallas_call(
    _combine_body,
    out_shape=jax.ShapeDtypeStruct((1, 1), jnp.float32),
    out_specs=pl.BlockSpec(memory_space=pltpu.SMEM),
)


def kernel(output_deltas, target_deltas, output_scores, target_scores):
    od = jnp.reshape(output_deltas, (_NS, _ND))
    td = jnp.reshape(target_deltas, (_NS, _ND))
    os_ = jnp.reshape(output_scores, (_NS, _NA))
    ts = jnp.reshape(target_scores, (_NS, _NA))
    inp = jnp.reshape(jnp.concatenate([od, td, os_, ts], axis=1), (-1,))
    part = _sc_partials(inp)
    return jnp.reshape(_combine(jnp.reshape(part, (_NS * 4, _L))), ())


# speedup vs baseline: 1.2857x; 1.2857x over previous
import functools

import jax
import jax.numpy as jnp
from jax import lax
from jax.experimental import pallas as pl
from jax.experimental.pallas import tpu as pltpu
from jax.experimental.pallas import tpu_sc as plsc

_EPS = 1e-7
_LN2 = 0.6931471805599453
_SQRT2 = 1.4142135623730951

_N = 36864
_NS, _L = 16, 16
_NA = _N // _NS
_ND = 4 * _NA
_TILE = 2 * _ND + 2 * _NA
_OD0, _TD0, _OS0, _TS0 = 0, _ND, 2 * _ND, 2 * _ND + _NA
_PAC = _TILE


def _log16(x):
    bits = lax.bitcast_convert_type(x, jnp.int32)
    e = lax.shift_right_logical(bits, 23) - 127
    m = lax.bitcast_convert_type((bits & 0x007FFFFF) | 0x3F800000, jnp.float32)
    big = m > _SQRT2
    m = jnp.where(big, m * 0.5, m)
    e = e + jnp.where(big, 1, 0)
    s = (m - 1.0) / (m + 1.0)
    z = s * s
    p = 1.0 + z * (1.0 / 3.0 + z * (1.0 / 5.0 + z * (1.0 / 7.0 + z * (1.0 / 9.0))))
    return e.astype(jnp.float32) * _LN2 + 2.0 * s * p


@functools.partial(
    pl.kernel,
    out_type=jax.ShapeDtypeStruct((_NS * 4 * _L,), jnp.float32),
    mesh=plsc.VectorSubcoreMesh(
        core_axis_name="c", subcore_axis_name="s",
        num_cores=1, num_subcores=_NS,
    ),
    scratch_types=[
        pltpu.VMEM((_TILE + 4 * _L,), jnp.float32),
        pltpu.SemaphoreType.DMA,
    ],
)
def _sc_partials(od_hbm, td_hbm, os_hbm, ts_hbm, part_hbm, buf_v, sem):
    sid = lax.axis_index("s")
    c0 = pltpu.async_copy(
        od_hbm.at[pl.ds(sid * _ND, _ND)], buf_v.at[pl.ds(_OD0, _ND)], sem)
    c1 = pltpu.async_copy(
        td_hbm.at[pl.ds(sid * _ND, _ND)], buf_v.at[pl.ds(_TD0, _ND)], sem)
    c2 = pltpu.async_copy(
        os_hbm.at[pl.ds(sid * _NA, _NA)], buf_v.at[pl.ds(_OS0, _NA)], sem)
    c3 = pltpu.async_copy(
        ts_hbm.at[pl.ds(sid * _NA, _NA)], buf_v.at[pl.ds(_TS0, _NA)], sem)
    c0.wait()
    c1.wait()
    c2.wait()
    c3.wait()

    zeros = jnp.zeros((_L,), jnp.float32)
    lane = lax.broadcasted_iota(jnp.int32, (_L,), 0)

    def body(k, carry):
        bce_a, nv_a, ps_a, reg_a = carry
        o_raw = buf_v[pl.ds(_OS0 + k * _L, _L)]
        t = buf_v[pl.ds(_TS0 + k * _L, _L)]
        o = jnp.clip(o_raw, _EPS, 1.0 - _EPS)
        bce = -(t * _log16(o) + (1.0 - t) * _log16(1.0 - o))
        valid = t != -1.0
        bce_a = bce_a + jnp.where(valid, bce, 0.0)
        nv_a = nv_a + jnp.where(valid, 1.0, 0.0)
        ps_a = ps_a + jnp.where(o_raw > 0.0, 1.0, 0.0)
        for v in range(4):
            dbase = k * (4 * _L) + v * _L
            od16 = buf_v[pl.ds(_OD0 + dbase, _L)]
            td16 = buf_v[pl.ds(_TD0 + dbase, _L)]
            d = jnp.abs(od16 - td16)
            sl1 = jnp.where(d < 1.0, 0.5 * d * d, d - 0.5)
            w = jnp.where(lane >= 12, o_raw[4 * v + 3],
                          jnp.where(lane >= 8, o_raw[4 * v + 2],
                                    jnp.where(lane >= 4, o_raw[4 * v + 1],
                                              o_raw[4 * v])))
            reg_a = reg_a + jnp.where(w > 0.0, sl1, 0.0)
        return bce_a, nv_a, ps_a, reg_a

    bce_a, nv_a, ps_a, reg_a = lax.fori_loop(
        0, _NA // _L, body, (zeros, zeros, zeros, zeros))

    buf_v[pl.ds(_PAC, _L)] = bce_a
    buf_v[pl.ds(_PAC + _L, _L)] = nv_a
    buf_v[pl.ds(_PAC + 2 * _L, _L)] = reg_a
    buf_v[pl.ds(_PAC + 3 * _L, _L)] = ps_a
    pltpu.sync_copy(
        buf_v.at[pl.ds(_PAC, 4 * _L)],
        part_hbm.at[pl.ds(sid * 4 * _L, 4 * _L)])


def _combine_body(p_ref, o_ref):
    p = p_ref[...]
    aid = lax.rem(lax.broadcasted_iota(jnp.int32, p.shape, 0), 4)
    bce = jnp.sum(jnp.where(aid == 0, p, 0.0))
    nv = jnp.sum(jnp.where(aid == 1, p, 0.0))
    reg = jnp.sum(jnp.where(aid == 2, p, 0.0))
    ps = jnp.sum(jnp.where(aid == 3, p, 0.0))
    cls_loss = bce / jnp.maximum(nv, 1.0)
    reg_loss = 10.0 * (reg / jnp.maximum(_EPS, ps))
    o_ref[0, 0] = cls_loss + reg_loss


_combine = pl.pallas_call(
    _combine_body,
    out_shape=jax.ShapeDtypeStruct((1, 1), jnp.float32),
    out_specs=pl.BlockSpec(memory_space=pltpu.SMEM),
)


def kernel(output_deltas, target_deltas, output_scores, target_scores):
    od = jnp.reshape(output_deltas, (-1,))
    td = jnp.reshape(target_deltas, (-1,))
    os_ = jnp.reshape(output_scores, (-1,))
    ts = jnp.reshape(target_scores, (-1,))
    part = _sc_partials(od, td, os_, ts)
    return jnp.reshape(_combine(jnp.reshape(part, (_NS * 4, _L))), ())
